# baseline (speedup 1.0000x reference)
import functools
import numpy as np
import jax
import jax.numpy as jnp
from jax import lax
from jax.experimental import pallas as pl
from jax.experimental.pallas import tpu as pltpu
from jax.experimental.pallas import tpu_sc as plsc

_MASKING_RATE = 0.4
_MSK_SCALAR = 0.0
_CHUNK = 256
_SC_WORKERS = 32
_SC_ROWS_PER_CHUNK = 16


def _mask_kernel(n_mask, r_ref, m_ref, k16_ref):
    i = pl.program_id(0)
    t = r_ref.shape[1]
    r = r_ref[0, :]
    rows = r_ref[0, pl.ds(i * _CHUNK, _CHUNK)]
    rj = r[None, :]
    rt = rows[:, None]
    jidx = lax.broadcasted_iota(jnp.int32, (_CHUNK, t), 1)
    tidx = i * _CHUNK + lax.broadcasted_iota(jnp.int32, (_CHUNK, t), 0)
    before = (rj < rt) | ((rj == rt) & (jidx < tidx))
    ranks = jnp.sum(before.astype(jnp.int32), axis=1, keepdims=True)
    masked = ranks < n_mask
    keep = jnp.where(masked, 0.0, 1.0)
    m_ref[0, :] = 1.0 - keep[:, 0]
    k16_ref[...] = jnp.broadcast_to(keep, (_CHUNK, 16))


def _tc_select_kernel(n_mask, r_ref, x1_ref, x2_ref, o1_ref, o2_ref):
    ti = pl.program_id(1)
    t = r_ref.shape[1]
    tb = x1_ref.shape[1]
    r = r_ref[0, :]
    rows = r_ref[0, pl.ds(ti * tb, tb)]
    rj = r[None, :]
    rt = rows[:, None]
    jidx = lax.broadcasted_iota(jnp.int32, (tb, t), 1)
    tidx = ti * tb + lax.broadcasted_iota(jnp.int32, (tb, t), 0)
    before = (rj < rt) | ((rj == rt) & (jidx < tidx))
    ranks = jnp.sum(before.astype(jnp.int32), axis=1, keepdims=True)
    masked = ranks < n_mask
    o1_ref[0] = jnp.where(masked, _MSK_SCALAR, x1_ref[0])
    o2_ref[0] = jnp.where(masked, _MSK_SCALAR, x2_ref[0])


def _sc_select_body(n_rows, t, f, x_ref, k16_ref, o_ref,
                    xin0, xin1, xout0, xout1, kbuf,
                    si0, si1, so0, so1):
    cc = lax.axis_index("c")
    ss = lax.axis_index("s")
    wid = ss * 2 + cc
    rpw = n_rows // _SC_WORKERS
    c = _SC_ROWS_PER_CHUNK
    nchunks = rpw // c
    base_row = wid * rpw
    t0 = lax.rem(base_row, t)
    pltpu.sync_copy(k16_ref.at[pl.ds(t0 * 16, rpw * 16)], kbuf)

    xins = (xin0, xin1)
    xouts = (xout0, xout1)
    sis = (si0, si1)
    sos = (so0, so1)

    def fill(g, b):
        return pltpu.make_async_copy(
            x_ref.at[pl.ds(base_row + g * c, c), :], xins[b], sis[b])

    def drain(g, b):
        return pltpu.make_async_copy(
            xouts[b], o_ref.at[pl.ds(base_row + g * c, c), :], sos[b])

    def compute(g, b):
        xin = xins[b]
        xout = xouts[b]

        def row_body(i2, carry):
            kv = kbuf[pl.ds((g * c + i2) * 16, 16)]
            for p in range(f // 16):
                xout[i2, pl.ds(p * 16, 16)] = xin[i2, pl.ds(p * 16, 16)] * kv
            return carry

        lax.fori_loop(0, c, row_body, 0)

    fill(0, 0).start()
    fill(1, 1).start()
    for b in range(2):
        fill(b, b).wait()
        compute(b, b)
        drain(b, b).start()
        fill(b + 2, b).start()

    def outer(g2, carry):
        for b in range(2):
            g = g2 * 2 + b
            fill(g, b).wait()
            drain(g - 2, b).wait()
            compute(g, b)
            drain(g, b).start()

            @pl.when(g + 2 < nchunks)
            def _():
                fill(g + 2, b).start()

        return carry

    lax.fori_loop(1, nchunks // 2, outer, 0)
    drain(nchunks - 2, 0).wait()
    drain(nchunks - 1, 1).wait()


def kernel(x_tre, x_sea, x_res):
    b, t, f = x_tre.shape
    n_mask = int(np.ceil(t * _MASKING_RATE))
    rk = jax.random.key(42)
    r = jax.random.uniform(rk, (t,), minval=0.0, maxval=1.0)

    mask, keep16 = pl.pallas_call(
        functools.partial(_mask_kernel, n_mask),
        grid=(t // _CHUNK,),
        in_specs=[pl.BlockSpec((1, t), lambda i: (0, 0))],
        out_specs=[pl.BlockSpec((1, _CHUNK), lambda i: (0, i)),
                   pl.BlockSpec((_CHUNK, 16), lambda i: (i, 0))],
        out_shape=[jax.ShapeDtypeStruct((1, t), jnp.float32),
                   jax.ShapeDtypeStruct((t, 16), jnp.float32)],
    )(r[None, :])

    n_rows = b * t
    c = _SC_ROWS_PER_CHUNK
    mesh = plsc.VectorSubcoreMesh(core_axis_name="c", subcore_axis_name="s")
    sc_select = pl.kernel(
        functools.partial(_sc_select_body, n_rows, t, f),
        out_type=jax.ShapeDtypeStruct((n_rows, f), jnp.float32),
        mesh=mesh,
        scratch_types=[
            pltpu.VMEM((c, f), jnp.float32),
            pltpu.VMEM((c, f), jnp.float32),
            pltpu.VMEM((c, f), jnp.float32),
            pltpu.VMEM((c, f), jnp.float32),
            pltpu.VMEM((n_rows // _SC_WORKERS * 16,), jnp.float32),
            pltpu.SemaphoreType.DMA,
            pltpu.SemaphoreType.DMA,
            pltpu.SemaphoreType.DMA,
            pltpu.SemaphoreType.DMA,
        ],
        compiler_params=pltpu.CompilerParams(use_tc_tiling_on_sc=True),
        cost_estimate=pl.CostEstimate(
            flops=n_rows * f,
            bytes_accessed=2 * n_rows * f * 4,
            transcendentals=0,
        ),
    )
    z_res = sc_select(x_res.reshape(n_rows, f), keep16.reshape(t * 16))
    z_res = z_res.reshape(b, t, f)

    tb = 1024
    x_spec = pl.BlockSpec((1, tb, f), lambda bi, ti: (bi, ti, 0))
    r_spec = pl.BlockSpec((1, t), lambda bi, ti: (0, 0))
    z_tre, z_sea = pl.pallas_call(
        functools.partial(_tc_select_kernel, n_mask),
        grid=(b, t // tb),
        in_specs=[r_spec, x_spec, x_spec],
        out_specs=[x_spec, x_spec],
        out_shape=[jax.ShapeDtypeStruct((b, t, f), jnp.float32)] * 2,
        compiler_params=pltpu.CompilerParams(
            dimension_semantics=("arbitrary", "arbitrary"),
        ),
    )(r[None, :], x_tre, x_sea)

    return (z_tre, z_sea, z_res, mask[0] != 0.0)


# speedup vs baseline: 1.3384x; 1.3384x over previous
import numpy as np
import jax
import jax.numpy as jnp
from jax.experimental import pallas as pl
from jax.experimental.pallas import tpu as pltpu

_MASKING_RATE = 0.4
_MSK_SCALAR = 0.0


def _fused_kernel(n_mask, r_ref, x1_ref, x2_ref, x3_ref,
                  o1_ref, o2_ref, o3_ref, m_ref, mscr_ref):
    ti = pl.program_id(0)
    bi = pl.program_id(1)
    t = r_ref.shape[1]
    tb = x1_ref.shape[1]

    @pl.when(bi == 0)
    def _():
        r = r_ref[0, :]
        rows = r_ref[0, pl.ds(ti * tb, tb)]
        rj = r[None, :]
        rt = rows[:, None]
        jidx = jax.lax.broadcasted_iota(jnp.int32, (tb, t), 1)
        tidx = ti * tb + jax.lax.broadcasted_iota(jnp.int32, (tb, t), 0)
        before = (rj < rt) | ((rj == rt) & (jidx < tidx))
        ranks = jnp.sum(before.astype(jnp.int32), axis=1, keepdims=True)
        mcol = (ranks < n_mask).astype(jnp.float32)
        mscr_ref[...] = mcol
        m_ref[0, :] = mcol[:, 0]

    masked = mscr_ref[...] != 0.0
    o1_ref[0] = jnp.where(masked, _MSK_SCALAR, x1_ref[0])
    o2_ref[0] = jnp.where(masked, _MSK_SCALAR, x2_ref[0])
    o3_ref[0] = jnp.where(masked, _MSK_SCALAR, x3_ref[0])


def kernel(x_tre, x_sea, x_res):
    b, t, f = x_tre.shape
    n_mask = int(np.ceil(t * _MASKING_RATE))
    rk = jax.random.key(42)
    r = jax.random.uniform(rk, (t,), minval=0.0, maxval=1.0)

    tb = 1024
    x_spec = pl.BlockSpec((1, tb, f), lambda ti, bi: (bi, ti, 0))
    r_spec = pl.BlockSpec((1, t), lambda ti, bi: (0, 0))
    m_spec = pl.BlockSpec((1, tb), lambda ti, bi: (0, ti))
    z_tre, z_sea, z_res, mask = pl.pallas_call(
        lambda *refs: _fused_kernel(n_mask, *refs),
        grid=(t // tb, b),
        in_specs=[r_spec, x_spec, x_spec, x_spec],
        out_specs=[x_spec, x_spec, x_spec, m_spec],
        out_shape=[jax.ShapeDtypeStruct((b, t, f), jnp.float32)] * 3
        + [jax.ShapeDtypeStruct((1, t), jnp.float32)],
        scratch_shapes=[pltpu.VMEM((tb, 1), jnp.float32)],
        compiler_params=pltpu.CompilerParams(
            dimension_semantics=("arbitrary", "arbitrary"),
        ),
    )(r[None, :], x_tre, x_sea, x_res)

    return (z_tre, z_sea, z_res, mask[0] != 0.0)
